# Initial kernel scaffold; baseline (speedup 1.0000x reference)
import jax
import jax.numpy as jnp
from jax.experimental import pallas as pl


def kernel(x, edge_index, Wq, bq, Wk, bk, Wv, bv, Wo, bo, W1, b1, W2, b2):
    raise NotImplementedError("write your pallas kernel here")


# speedup vs baseline: 19.3510x; 19.3510x over previous
import dataclasses
import functools
import math

import jax
import jax.numpy as jnp
from jax import lax
from jax.experimental import pallas as pl
from jax.experimental.pallas import tpu as pltpu
from jax.experimental.pallas import tpu_sc as plsc

_B, _N, _D, _E, _H = 4, 1024, 512, 16384, 8
_HID = _D // 2
_HD = _D // _H
_KTOP = max(1, int(_N * 0.5))
_NQ = 128

_SC_NC, _SC_NS, _SC_L = 2, 16, 16
_NW = _SC_NC * _SC_NS
_RPW = _N // _NW


def _build_counts(rows, cols):
    mesh = plsc.VectorSubcoreMesh(core_axis_name="c", subcore_axis_name="s")
    cp = pltpu.CompilerParams()
    if "needs_layout_passes" in pltpu.CompilerParams.__dataclass_fields__:
        cp = dataclasses.replace(cp, needs_layout_passes=False)

    @functools.partial(
        pl.kernel,
        out_type=jax.ShapeDtypeStruct((_N, _N), jnp.float32),
        mesh=mesh,
        compiler_params=cp,
        scratch_types=[
            pltpu.VMEM((_RPW, _N), jnp.float32),
            pltpu.VMEM((_E,), jnp.int32),
            pltpu.VMEM((_E,), jnp.int32),
        ],
    )
    def sc_kernel(r_hbm, c_hbm, a_hbm, a_v, r_v, c_v):
        wid = lax.axis_index("s") * _SC_NC + lax.axis_index("c")
        lo = wid * _RPW
        zero = jnp.zeros((_SC_L,), jnp.float32)

        @pl.loop(0, _RPW)
        def _(i):
            @pl.loop(0, _N, step=_SC_L)
            def _(j):
                a_v[i, pl.ds(j, _SC_L)] = zero

        pltpu.sync_copy(r_hbm, r_v)
        pltpu.sync_copy(c_hbm, c_v)
        ones = jnp.ones((_SC_L,), jnp.float32)

        @pl.loop(0, _E, step=_SC_L)
        def _(e):
            r = r_v[pl.ds(e, _SC_L)]
            c = c_v[pl.ds(e, _SC_L)]
            m = (r >= lo) & (r < lo + _RPW)
            ri = jnp.where(m, r - lo, 0)
            ci = jnp.where(m, c, 0)
            plsc.addupdate_scatter(a_v, [ri, ci], ones, mask=m)

        pltpu.sync_copy(a_v, a_hbm.at[pl.ds(lo, _RPW)])

    return sc_kernel(rows, cols)


def _topo_body(a_ref, x_ref, w1t_ref, b1_ref, w2_ref, s_ref):
    xb = x_ref[0]
    aggr = jnp.dot(a_ref[...], xb, preferred_element_type=jnp.float32,
                   precision=lax.Precision.HIGHEST)
    h = jnp.maximum(
        jnp.dot(aggr, w1t_ref[...], preferred_element_type=jnp.float32)
        + b1_ref[...],
        0.0,
    )
    s_ref[0] = lax.dot_general(
        w2_ref[...], h, (((1,), (1,)), ((), ())),
        preferred_element_type=jnp.float32,
    )


def _topo(a, x, w1t, b1r, w2):
    return pl.pallas_call(
        _topo_body,
        grid=(_B,),
        in_specs=[
            pl.BlockSpec((_N, _N), lambda b: (0, 0)),
            pl.BlockSpec((1, _N, _D), lambda b: (b, 0, 0)),
            pl.BlockSpec((_D, _HID), lambda b: (0, 0)),
            pl.BlockSpec((1, _HID), lambda b: (0, 0)),
            pl.BlockSpec((1, _HID), lambda b: (0, 0)),
        ],
        out_specs=pl.BlockSpec((1, 1, _N), lambda b: (b, 0, 0)),
        out_shape=jax.ShapeDtypeStruct((_B, 1, _N), jnp.float32),
    )(a, x, w1t, b1r, w2)


def _colmask_body(sr_ref, sc_ref, m_ref):
    sr = sr_ref[0]
    sc = sc_ref[0]
    ii = lax.broadcasted_iota(jnp.int32, (_N, _N), 0)
    jj = lax.broadcasted_iota(jnp.int32, (_N, _N), 1)
    gt = (sc > sr).astype(jnp.float32)
    eqb = ((sc == sr) & (ii < jj)).astype(jnp.float32)
    rank = jnp.sum(gt + eqb, axis=0, keepdims=True)
    m_ref[0] = (rank < float(_KTOP)).astype(jnp.float32)


def _colmask(scores_row, scores_col):
    return pl.pallas_call(
        _colmask_body,
        grid=(_B,),
        in_specs=[
            pl.BlockSpec((1, 1, _N), lambda b: (b, 0, 0)),
            pl.BlockSpec((1, _N, 1), lambda b: (b, 0, 0)),
        ],
        out_specs=pl.BlockSpec((1, 1, _N), lambda b: (b, 0, 0)),
        out_shape=jax.ShapeDtypeStruct((_B, 1, _N), jnp.float32),
    )(scores_row, scores_col)


def _attn_body(x_ref, a_ref, cm_ref,
               wqt_ref, bq_ref, wkt_ref, bk_ref, wvt_ref, bv_ref,
               wot_ref, bo_ref,
               out_ref, mask_ref, k_s, v_s):
    iq = pl.program_id(1)
    xb = x_ref[0]

    @pl.when(iq == 0)
    def _():
        k_s[...] = (
            jnp.dot(xb, wkt_ref[...], preferred_element_type=jnp.float32)
            + bk_ref[...]
        )
        v_s[...] = (
            jnp.dot(xb, wvt_ref[...], preferred_element_type=jnp.float32)
            + bv_ref[...]
        )

    xq = x_ref[0, pl.ds(iq * _NQ, _NQ), :]
    q = (
        jnp.dot(xq, wqt_ref[...], preferred_element_type=jnp.float32)
        + bq_ref[...]
    )

    edge = (a_ref[...] > 0.0).astype(jnp.float32)
    mask = jnp.maximum(edge, cm_ref[0])

    scale = 1.0 / math.sqrt(_HD)
    outs = []
    for h in range(_H):
        qh = q[:, h * _HD:(h + 1) * _HD]
        kh = k_s[:, h * _HD:(h + 1) * _HD]
        vh = v_s[:, h * _HD:(h + 1) * _HD]
        s = lax.dot_general(
            qh, kh, (((1,), (1,)), ((), ())),
            preferred_element_type=jnp.float32,
        ) * scale
        m = jnp.max(s, axis=1, keepdims=True)
        e = jnp.exp(s - m)
        z = jnp.sum(e, axis=1, keepdims=True)
        em = e * mask
        zm = jnp.sum(em, axis=1, keepdims=True)
        oh = jnp.dot(em, vh, preferred_element_type=jnp.float32)
        outs.append(oh / (zm + 1e-8 * z))
    o = jnp.concatenate(outs, axis=1)
    out_ref[0] = (
        jnp.dot(o, wot_ref[...], preferred_element_type=jnp.float32)
        + bo_ref[...]
    )
    mask_ref[0] = jnp.broadcast_to(mask[None, :, :], (_H, _NQ, _N))


def _attention(x, a, cm, wqt, bq, wkt, bk, wvt, bv, wot, bo):
    return pl.pallas_call(
        _attn_body,
        grid=(_B, _N // _NQ),
        in_specs=[
            pl.BlockSpec((1, _N, _D), lambda b, i: (b, 0, 0)),
            pl.BlockSpec((_NQ, _N), lambda b, i: (i, 0)),
            pl.BlockSpec((1, 1, _N), lambda b, i: (b, 0, 0)),
            pl.BlockSpec((_D, _D), lambda b, i: (0, 0)),
            pl.BlockSpec((1, _D), lambda b, i: (0, 0)),
            pl.BlockSpec((_D, _D), lambda b, i: (0, 0)),
            pl.BlockSpec((1, _D), lambda b, i: (0, 0)),
            pl.BlockSpec((_D, _D), lambda b, i: (0, 0)),
            pl.BlockSpec((1, _D), lambda b, i: (0, 0)),
            pl.BlockSpec((_D, _D), lambda b, i: (0, 0)),
            pl.BlockSpec((1, _D), lambda b, i: (0, 0)),
        ],
        out_specs=[
            pl.BlockSpec((1, _NQ, _D), lambda b, i: (b, i, 0)),
            pl.BlockSpec((1, _H, _NQ, _N), lambda b, i: (b, 0, i, 0)),
        ],
        out_shape=[
            jax.ShapeDtypeStruct((_B, _N, _D), jnp.float32),
            jax.ShapeDtypeStruct((_B, _H, _N, _N), jnp.float32),
        ],
        scratch_shapes=[
            pltpu.VMEM((_N, _D), jnp.float32),
            pltpu.VMEM((_N, _D), jnp.float32),
        ],
    )(x, a, cm, wqt, bq, wkt, bk, wvt, bv, wot, bo)


def kernel(x, edge_index, Wq, bq, Wk, bk, Wv, bv, Wo, bo, W1, b1, W2, b2):
    a = _build_counts(edge_index[0], edge_index[1])
    scores = _topo(a, x, W1.T, b1.reshape(1, _HID), W2)
    cm = _colmask(scores, scores.reshape(_B, _N, 1))
    out, mask_h = _attention(
        x, a, cm,
        Wq.T, bq.reshape(1, _D),
        Wk.T, bk.reshape(1, _D),
        Wv.T, bv.reshape(1, _D),
        Wo.T, bo.reshape(1, _D),
    )
    return out, mask_h


# speedup vs baseline: 29.2983x; 1.5140x over previous
import dataclasses
import functools
import math

import jax
import jax.numpy as jnp
from jax import lax
from jax.experimental import pallas as pl
from jax.experimental.pallas import tpu as pltpu
from jax.experimental.pallas import tpu_sc as plsc

_B, _N, _D, _E, _H = 4, 1024, 512, 16384, 8
_HID = _D // 2
_HD = _D // _H
_KTOP = max(1, int(_N * 0.5))
_NQ = 256

_SC_NC, _SC_NS, _SC_L = 2, 16, 16
_NW = _SC_NC * _SC_NS
_RPW = _N // _NW


def _build_counts(rows, cols):
    mesh = plsc.VectorSubcoreMesh(core_axis_name="c", subcore_axis_name="s")
    cp = pltpu.CompilerParams()
    if "needs_layout_passes" in pltpu.CompilerParams.__dataclass_fields__:
        cp = dataclasses.replace(cp, needs_layout_passes=False)

    @functools.partial(
        pl.kernel,
        out_type=jax.ShapeDtypeStruct((_N, _N), jnp.float32),
        mesh=mesh,
        compiler_params=cp,
        scratch_types=[
            pltpu.VMEM((_RPW, _N), jnp.float32),
            pltpu.VMEM((_E,), jnp.int32),
            pltpu.VMEM((_E,), jnp.int32),
        ],
    )
    def sc_kernel(r_hbm, c_hbm, a_hbm, a_v, r_v, c_v):
        wid = lax.axis_index("s") * _SC_NC + lax.axis_index("c")
        lo = wid * _RPW
        zero = jnp.zeros((_SC_L,), jnp.float32)

        @pl.loop(0, _RPW)
        def _(i):
            @pl.loop(0, _N, step=_SC_L)
            def _(j):
                a_v[i, pl.ds(j, _SC_L)] = zero

        pltpu.sync_copy(r_hbm, r_v)
        pltpu.sync_copy(c_hbm, c_v)
        ones = jnp.ones((_SC_L,), jnp.float32)

        @pl.loop(0, _E, step=_SC_L)
        def _(e):
            r = r_v[pl.ds(e, _SC_L)]
            c = c_v[pl.ds(e, _SC_L)]
            m = (r >= lo) & (r < lo + _RPW)
            ri = jnp.where(m, r - lo, 0)
            ci = jnp.where(m, c, 0)
            plsc.addupdate_scatter(a_v, [ri, ci], ones, mask=m)

        pltpu.sync_copy(a_v, a_hbm.at[pl.ds(lo, _RPW)])

    return sc_kernel(rows, cols)


def _proj_body(x_ref, wqt_ref, bq_ref, wk_ref, bkc_ref, wv_ref, bvc_ref,
               q_ref, kt_ref, vt_ref):
    xb = x_ref[0]
    scale = 1.0 / math.sqrt(_HD)
    q = (jnp.dot(xb, wqt_ref[...], preferred_element_type=jnp.float32)
         + bq_ref[...]) * scale
    q_ref[0] = q.astype(jnp.bfloat16)
    kt = lax.dot_general(wk_ref[...], xb, (((1,), (1,)), ((), ())),
                         preferred_element_type=jnp.float32) + bkc_ref[...]
    kt_ref[0] = kt.astype(jnp.bfloat16)
    vt = lax.dot_general(wv_ref[...], xb, (((1,), (1,)), ((), ())),
                         preferred_element_type=jnp.float32) + bvc_ref[...]
    vt_ref[0] = vt.astype(jnp.bfloat16)


def _proj(x, wqt, bq, wk, bkc, wv, bvc):
    return pl.pallas_call(
        _proj_body,
        grid=(_B,),
        in_specs=[
            pl.BlockSpec((1, _N, _D), lambda b: (b, 0, 0)),
            pl.BlockSpec((_D, _D), lambda b: (0, 0)),
            pl.BlockSpec((1, _D), lambda b: (0, 0)),
            pl.BlockSpec((_D, _D), lambda b: (0, 0)),
            pl.BlockSpec((_D, 1), lambda b: (0, 0)),
            pl.BlockSpec((_D, _D), lambda b: (0, 0)),
            pl.BlockSpec((_D, 1), lambda b: (0, 0)),
        ],
        out_specs=[
            pl.BlockSpec((1, _N, _D), lambda b: (b, 0, 0)),
            pl.BlockSpec((1, _D, _N), lambda b: (b, 0, 0)),
            pl.BlockSpec((1, _D, _N), lambda b: (b, 0, 0)),
        ],
        out_shape=[
            jax.ShapeDtypeStruct((_B, _N, _D), jnp.bfloat16),
            jax.ShapeDtypeStruct((_B, _D, _N), jnp.bfloat16),
            jax.ShapeDtypeStruct((_B, _D, _N), jnp.bfloat16),
        ],
    )(x, wqt, bq, wk, bkc, wv, bvc)


def _topo_body(a_ref, x_ref, w1t_ref, b1_ref, w2_ref, s_ref):
    xb = x_ref[0]
    ab = a_ref[...].astype(jnp.bfloat16)
    x1 = xb.astype(jnp.bfloat16)
    r1 = xb - x1.astype(jnp.float32)
    x2 = r1.astype(jnp.bfloat16)
    x3 = (r1 - x2.astype(jnp.float32)).astype(jnp.bfloat16)
    aggr = (
        jnp.dot(ab, x1, preferred_element_type=jnp.float32)
        + jnp.dot(ab, x2, preferred_element_type=jnp.float32)
        + jnp.dot(ab, x3, preferred_element_type=jnp.float32)
    )
    h = jnp.maximum(
        jnp.dot(aggr, w1t_ref[...], preferred_element_type=jnp.float32)
        + b1_ref[...],
        0.0,
    )
    s_ref[0] = lax.dot_general(
        w2_ref[...], h, (((1,), (1,)), ((), ())),
        preferred_element_type=jnp.float32,
    )


def _topo(a, x, w1t, b1r, w2):
    return pl.pallas_call(
        _topo_body,
        grid=(_B,),
        in_specs=[
            pl.BlockSpec((_N, _N), lambda b: (0, 0)),
            pl.BlockSpec((1, _N, _D), lambda b: (b, 0, 0)),
            pl.BlockSpec((_D, _HID), lambda b: (0, 0)),
            pl.BlockSpec((1, _HID), lambda b: (0, 0)),
            pl.BlockSpec((1, _HID), lambda b: (0, 0)),
        ],
        out_specs=pl.BlockSpec((1, 1, _N), lambda b: (b, 0, 0)),
        out_shape=jax.ShapeDtypeStruct((_B, 1, _N), jnp.float32),
    )(a, x, w1t, b1r, w2)


def _colmask_body(sr_ref, sc_ref, m_ref):
    sr = sr_ref[0]
    sc = sc_ref[0]
    ii = lax.broadcasted_iota(jnp.int32, (_N, _N), 0)
    jj = lax.broadcasted_iota(jnp.int32, (_N, _N), 1)
    gt = (sc > sr).astype(jnp.float32)
    eqb = ((sc == sr) & (ii < jj)).astype(jnp.float32)
    rank = jnp.sum(gt + eqb, axis=0, keepdims=True)
    m_ref[0] = (rank < float(_KTOP)).astype(jnp.float32)


def _colmask(scores_row, scores_col):
    return pl.pallas_call(
        _colmask_body,
        grid=(_B,),
        in_specs=[
            pl.BlockSpec((1, 1, _N), lambda b: (b, 0, 0)),
            pl.BlockSpec((1, _N, 1), lambda b: (b, 0, 0)),
        ],
        out_specs=pl.BlockSpec((1, 1, _N), lambda b: (b, 0, 0)),
        out_shape=jax.ShapeDtypeStruct((_B, 1, _N), jnp.float32),
    )(scores_row, scores_col)


def _attn_body(q_ref, kt_ref, vt_ref, a_ref, cm_ref, wot_ref, bo_ref,
               out_ref, mask_ref):
    edge = (a_ref[...] > 0.0).astype(jnp.float32)
    mask = jnp.maximum(edge, cm_ref[0])
    logm = jnp.where(mask > 0.0, 0.0, -jnp.inf)
    ones_c = jnp.ones((_N, 1), jnp.bfloat16)

    outs = []
    for h in range(_H):
        qh = q_ref[0][:, h * _HD:(h + 1) * _HD]
        kth = kt_ref[0, pl.ds(h * _HD, _HD), :]
        vth = vt_ref[0, pl.ds(h * _HD, _HD), :]
        s = jnp.dot(qh, kth, preferred_element_type=jnp.float32)
        em = jnp.exp(s + logm).astype(jnp.bfloat16)
        zm = jnp.dot(em, ones_c,
                     preferred_element_type=jnp.float32)
        oh = lax.dot_general(em, vth, (((1,), (1,)), ((), ())),
                             preferred_element_type=jnp.float32)
        outs.append(oh / zm)
    o = jnp.concatenate(outs, axis=1)
    out_ref[0] = (
        jnp.dot(o, wot_ref[...], preferred_element_type=jnp.float32)
        + bo_ref[...]
    )
    mask_ref[0] = jnp.broadcast_to(mask[None, :, :], (_H, _NQ, _N))


def _attention(q, kt, vt, a, cm, wot, bo):
    return pl.pallas_call(
        _attn_body,
        grid=(_B, _N // _NQ),
        in_specs=[
            pl.BlockSpec((1, _NQ, _D), lambda b, i: (b, i, 0)),
            pl.BlockSpec((1, _D, _N), lambda b, i: (b, 0, 0)),
            pl.BlockSpec((1, _D, _N), lambda b, i: (b, 0, 0)),
            pl.BlockSpec((_NQ, _N), lambda b, i: (i, 0)),
            pl.BlockSpec((1, 1, _N), lambda b, i: (b, 0, 0)),
            pl.BlockSpec((_D, _D), lambda b, i: (0, 0)),
            pl.BlockSpec((1, _D), lambda b, i: (0, 0)),
        ],
        out_specs=[
            pl.BlockSpec((1, _NQ, _D), lambda b, i: (b, i, 0)),
            pl.BlockSpec((1, _H, _NQ, _N), lambda b, i: (b, 0, i, 0)),
        ],
        out_shape=[
            jax.ShapeDtypeStruct((_B, _N, _D), jnp.float32),
            jax.ShapeDtypeStruct((_B, _H, _N, _N), jnp.float32),
        ],
    )(q, kt, vt, a, cm, wot, bo)


def kernel(x, edge_index, Wq, bq, Wk, bk, Wv, bv, Wo, bo, W1, b1, W2, b2):
    a = _build_counts(edge_index[0], edge_index[1])
    q, kt, vt = _proj(
        x,
        Wq.T, bq.reshape(1, _D),
        Wk, bk.reshape(_D, 1),
        Wv, bv.reshape(_D, 1),
    )
    scores = _topo(a, x, W1.T, b1.reshape(1, _HID), W2)
    cm = _colmask(scores, scores.reshape(_B, _N, 1))
    out, mask_h = _attention(q, kt, vt, a, cm, Wo.T, bo.reshape(1, _D))
    return out, mask_h


# speedup vs baseline: 31.7056x; 1.0822x over previous
import dataclasses
import functools
import math

import jax
import jax.numpy as jnp
from jax import lax
from jax.experimental import pallas as pl
from jax.experimental.pallas import tpu as pltpu
from jax.experimental.pallas import tpu_sc as plsc

_B, _N, _D, _E, _H = 4, 1024, 512, 16384, 8
_HID = _D // 2
_HD = _D // _H
_KTOP = max(1, int(_N * 0.5))
_NQ = 256

_SC_NC, _SC_NS, _SC_L = 2, 16, 16
_NW = _SC_NC * _SC_NS
_RPW = _N // _NW


def _build_counts(rows, cols):
    mesh = plsc.VectorSubcoreMesh(core_axis_name="c", subcore_axis_name="s")
    cp = pltpu.CompilerParams()
    if "needs_layout_passes" in pltpu.CompilerParams.__dataclass_fields__:
        cp = dataclasses.replace(cp, needs_layout_passes=False)

    @functools.partial(
        pl.kernel,
        out_type=jax.ShapeDtypeStruct((_N, _N), jnp.float32),
        mesh=mesh,
        compiler_params=cp,
        scratch_types=[
            pltpu.VMEM((_RPW, _N), jnp.float32),
            pltpu.VMEM((_E,), jnp.int32),
            pltpu.VMEM((_E,), jnp.int32),
        ],
    )
    def sc_kernel(r_hbm, c_hbm, a_hbm, a_v, r_v, c_v):
        wid = lax.axis_index("s") * _SC_NC + lax.axis_index("c")
        lo = wid * _RPW
        zero = jnp.zeros((_SC_L,), jnp.float32)

        @pl.loop(0, _RPW)
        def _(i):
            @pl.loop(0, _N, step=_SC_L)
            def _(j):
                a_v[i, pl.ds(j, _SC_L)] = zero

        pltpu.sync_copy(r_hbm, r_v)
        pltpu.sync_copy(c_hbm, c_v)
        ones = jnp.ones((_SC_L,), jnp.float32)

        @pl.loop(0, _E, step=_SC_L)
        def _(e):
            r = r_v[pl.ds(e, _SC_L)]
            c = c_v[pl.ds(e, _SC_L)]
            m = (r >= lo) & (r < lo + _RPW)
            ri = jnp.where(m, r - lo, 0)
            ci = jnp.where(m, c, 0)
            plsc.addupdate_scatter(a_v, [ri, ci], ones, mask=m)

        pltpu.sync_copy(a_v, a_hbm.at[pl.ds(lo, _RPW)])

    return sc_kernel(rows, cols)


def _proj_body(x_ref, wqt_ref, bq_ref, wk_ref, bkc_ref, wv_ref, bvc_ref,
               q_ref, kt_ref, vt_ref):
    xb = x_ref[0]
    scale = 1.0 / math.sqrt(_HD)
    q = (jnp.dot(xb, wqt_ref[...], preferred_element_type=jnp.float32)
         + bq_ref[...]) * scale
    q_ref[0] = q.astype(jnp.bfloat16)
    kt = lax.dot_general(wk_ref[...], xb, (((1,), (1,)), ((), ())),
                         preferred_element_type=jnp.float32) + bkc_ref[...]
    kt_ref[0] = kt.astype(jnp.bfloat16)
    vt = lax.dot_general(wv_ref[...], xb, (((1,), (1,)), ((), ())),
                         preferred_element_type=jnp.float32) + bvc_ref[...]
    vtb = vt.astype(jnp.bfloat16)
    pad = jnp.where(
        lax.broadcasted_iota(jnp.int32, (8, _N), 0) == 0,
        1.0, 0.0).astype(jnp.bfloat16)
    for h in range(_H):
        vt_ref[0, pl.ds(h * 72, _HD), :] = vtb[h * _HD:(h + 1) * _HD, :]
        vt_ref[0, pl.ds(h * 72 + _HD, 8), :] = pad


def _proj(x, wqt, bq, wk, bkc, wv, bvc):
    return pl.pallas_call(
        _proj_body,
        grid=(_B,),
        in_specs=[
            pl.BlockSpec((1, _N, _D), lambda b: (b, 0, 0)),
            pl.BlockSpec((_D, _D), lambda b: (0, 0)),
            pl.BlockSpec((1, _D), lambda b: (0, 0)),
            pl.BlockSpec((_D, _D), lambda b: (0, 0)),
            pl.BlockSpec((_D, 1), lambda b: (0, 0)),
            pl.BlockSpec((_D, _D), lambda b: (0, 0)),
            pl.BlockSpec((_D, 1), lambda b: (0, 0)),
        ],
        out_specs=[
            pl.BlockSpec((1, _N, _D), lambda b: (b, 0, 0)),
            pl.BlockSpec((1, _D, _N), lambda b: (b, 0, 0)),
            pl.BlockSpec((1, _H * 72, _N), lambda b: (b, 0, 0)),
        ],
        out_shape=[
            jax.ShapeDtypeStruct((_B, _N, _D), jnp.bfloat16),
            jax.ShapeDtypeStruct((_B, _D, _N), jnp.bfloat16),
            jax.ShapeDtypeStruct((_B, _H * 72, _N), jnp.bfloat16),
        ],
    )(x, wqt, bq, wk, bkc, wv, bvc)


def _topo_body(a_ref, x_ref, w1t_ref, b1_ref, w2_ref, s_ref):
    xb = x_ref[0]
    ab = a_ref[...].astype(jnp.bfloat16)
    x1 = xb.astype(jnp.bfloat16)
    r1 = xb - x1.astype(jnp.float32)
    x2 = r1.astype(jnp.bfloat16)
    x3 = (r1 - x2.astype(jnp.float32)).astype(jnp.bfloat16)
    aggr = (
        jnp.dot(ab, x1, preferred_element_type=jnp.float32)
        + jnp.dot(ab, x2, preferred_element_type=jnp.float32)
        + jnp.dot(ab, x3, preferred_element_type=jnp.float32)
    )
    h = jnp.maximum(
        jnp.dot(aggr, w1t_ref[...], preferred_element_type=jnp.float32)
        + b1_ref[...],
        0.0,
    )
    s_ref[0] = lax.dot_general(
        w2_ref[...], h, (((1,), (1,)), ((), ())),
        preferred_element_type=jnp.float32,
    )


def _topo(a, x, w1t, b1r, w2):
    return pl.pallas_call(
        _topo_body,
        grid=(_B,),
        in_specs=[
            pl.BlockSpec((_N, _N), lambda b: (0, 0)),
            pl.BlockSpec((1, _N, _D), lambda b: (b, 0, 0)),
            pl.BlockSpec((_D, _HID), lambda b: (0, 0)),
            pl.BlockSpec((1, _HID), lambda b: (0, 0)),
            pl.BlockSpec((1, _HID), lambda b: (0, 0)),
        ],
        out_specs=pl.BlockSpec((1, 1, _N), lambda b: (b, 0, 0)),
        out_shape=jax.ShapeDtypeStruct((_B, 1, _N), jnp.float32),
    )(a, x, w1t, b1r, w2)


def _colmask_body(sr_ref, sc_ref, m_ref):
    sr = sr_ref[0]
    sc = sc_ref[0]
    ii = lax.broadcasted_iota(jnp.int32, (_N, _N), 0)
    jj = lax.broadcasted_iota(jnp.int32, (_N, _N), 1)
    gt = (sc > sr).astype(jnp.float32)
    eqb = ((sc == sr) & (ii < jj)).astype(jnp.float32)
    rank = jnp.sum(gt + eqb, axis=0, keepdims=True)
    m_ref[0] = (rank < float(_KTOP)).astype(jnp.float32)


def _colmask(scores_row, scores_col):
    return pl.pallas_call(
        _colmask_body,
        grid=(_B,),
        in_specs=[
            pl.BlockSpec((1, 1, _N), lambda b: (b, 0, 0)),
            pl.BlockSpec((1, _N, 1), lambda b: (b, 0, 0)),
        ],
        out_specs=pl.BlockSpec((1, 1, _N), lambda b: (b, 0, 0)),
        out_shape=jax.ShapeDtypeStruct((_B, 1, _N), jnp.float32),
    )(scores_row, scores_col)


def _attn_body(q_ref, kt_ref, vt_ref, a_ref, cm_ref, wot_ref, bo_ref,
               out_ref, mask_ref):
    edge = (a_ref[...] > 0.0).astype(jnp.float32)
    mask = jnp.maximum(edge, cm_ref[0])
    mask_bf = mask.astype(jnp.bfloat16)

    outs = []
    for h in range(_H):
        qh = q_ref[0][:, h * _HD:(h + 1) * _HD]
        kth = kt_ref[0, pl.ds(h * _HD, _HD), :]
        vth = vt_ref[0, pl.ds(h * 72, 72), :]
        s = jnp.dot(qh, kth, preferred_element_type=jnp.float32)
        em = jnp.exp(s).astype(jnp.bfloat16) * mask_bf
        ohz = lax.dot_general(em, vth, (((1,), (1,)), ((), ())),
                              preferred_element_type=jnp.float32)
        outs.append(ohz[:, :_HD] * (1.0 / ohz[:, _HD:_HD + 1]))
    o = jnp.concatenate(outs, axis=1)
    out_ref[0] = (
        jnp.dot(o, wot_ref[...], preferred_element_type=jnp.float32)
        + bo_ref[...]
    )
    mask_ref[0] = jnp.broadcast_to(mask[None, :, :], (_H, _NQ, _N))


def _attention(q, kt, vt, a, cm, wot, bo):
    return pl.pallas_call(
        _attn_body,
        grid=(_B, _N // _NQ),
        in_specs=[
            pl.BlockSpec((1, _NQ, _D), lambda b, i: (b, i, 0)),
            pl.BlockSpec((1, _D, _N), lambda b, i: (b, 0, 0)),
            pl.BlockSpec((1, _H * 72, _N), lambda b, i: (b, 0, 0)),
            pl.BlockSpec((_NQ, _N), lambda b, i: (i, 0)),
            pl.BlockSpec((1, 1, _N), lambda b, i: (b, 0, 0)),
            pl.BlockSpec((_D, _D), lambda b, i: (0, 0)),
            pl.BlockSpec((1, _D), lambda b, i: (0, 0)),
        ],
        out_specs=[
            pl.BlockSpec((1, _NQ, _D), lambda b, i: (b, i, 0)),
            pl.BlockSpec((1, _H, _NQ, _N), lambda b, i: (b, 0, i, 0)),
        ],
        out_shape=[
            jax.ShapeDtypeStruct((_B, _N, _D), jnp.float32),
            jax.ShapeDtypeStruct((_B, _H, _N, _N), jnp.float32),
        ],
    )(q, kt, vt, a, cm, wot, bo)


def kernel(x, edge_index, Wq, bq, Wk, bk, Wv, bv, Wo, bo, W1, b1, W2, b2):
    a = _build_counts(edge_index[0], edge_index[1])
    q, kt, vt = _proj(
        x,
        Wq.T, bq.reshape(1, _D),
        Wk, bk.reshape(_D, 1),
        Wv, bv.reshape(_D, 1),
    )
    scores = _topo(a, x, W1.T, b1.reshape(1, _HID), W2)
    cm = _colmask(scores, scores.reshape(_B, _N, 1))
    out, mask_h = _attention(q, kt, vt, a, cm, Wo.T, bo.reshape(1, _D))
    return out, mask_h


# speedup vs baseline: 34.7307x; 1.0954x over previous
import dataclasses
import functools
import math

import jax
import jax.numpy as jnp
from jax import lax
from jax.experimental import pallas as pl
from jax.experimental.pallas import tpu as pltpu
from jax.experimental.pallas import tpu_sc as plsc

_B, _N, _D, _E, _H = 4, 1024, 512, 16384, 8
_HID = _D // 2
_HD = _D // _H
_KTOP = max(1, int(_N * 0.5))
_NQ = 256

_SC_NC, _SC_NS, _SC_L = 2, 16, 16
_NW = _SC_NC * _SC_NS
_RPW = _N // _NW


def _build_counts(rows, cols):
    mesh = plsc.VectorSubcoreMesh(core_axis_name="c", subcore_axis_name="s")
    cp = pltpu.CompilerParams()
    if "needs_layout_passes" in pltpu.CompilerParams.__dataclass_fields__:
        cp = dataclasses.replace(cp, needs_layout_passes=False)

    @functools.partial(
        pl.kernel,
        out_type=jax.ShapeDtypeStruct((_N, _N), jnp.float32),
        mesh=mesh,
        compiler_params=cp,
        scratch_types=[
            pltpu.VMEM((_RPW, _N), jnp.float32),
            pltpu.VMEM((_E,), jnp.int32),
            pltpu.VMEM((_E,), jnp.int32),
            pltpu.SemaphoreType.DMA,
            pltpu.SemaphoreType.DMA,
        ],
    )
    def sc_kernel(r_hbm, c_hbm, a_hbm, a_v, r_v, c_v, sem_r, sem_c):
        wid = lax.axis_index("s") * _SC_NC + lax.axis_index("c")
        lo = wid * _RPW
        zero = jnp.zeros((_SC_L,), jnp.float32)

        cp_r = pltpu.async_copy(r_hbm, r_v, sem_r)
        cp_c = pltpu.async_copy(c_hbm, c_v, sem_c)

        @pl.loop(0, _RPW)
        def _(i):
            @plsc.parallel_loop(0, _N, _SC_L, unroll=8)
            def _(j):
                a_v[i, pl.ds(j, _SC_L)] = zero

        cp_r.wait()
        cp_c.wait()
        ones = jnp.ones((_SC_L,), jnp.float32)

        @plsc.parallel_loop(0, _E, _SC_L, unroll=4)
        def _(e):
            r = r_v[pl.ds(e, _SC_L)]
            c = c_v[pl.ds(e, _SC_L)]
            m = (r >= lo) & (r < lo + _RPW)
            ri = jnp.where(m, r - lo, 0)
            ci = jnp.where(m, c, 0)
            plsc.addupdate_scatter(a_v, [ri, ci], ones, mask=m)

        pltpu.sync_copy(a_v, a_hbm.at[pl.ds(lo, _RPW)])

    return sc_kernel(rows, cols)


def _proj_body(x_ref, wqt_ref, bq_ref, wk_ref, bkc_ref, wv_ref, bvc_ref,
               q_ref, kt_ref, vt_ref):
    xb = x_ref[0]
    scale = 1.0 / math.sqrt(_HD)
    q = (jnp.dot(xb, wqt_ref[...], preferred_element_type=jnp.float32)
         + bq_ref[...]) * scale
    q_ref[0] = q.astype(jnp.bfloat16)
    kt = lax.dot_general(wk_ref[...], xb, (((1,), (1,)), ((), ())),
                         preferred_element_type=jnp.float32) + bkc_ref[...]
    kt_ref[0] = kt.astype(jnp.bfloat16)
    vt = lax.dot_general(wv_ref[...], xb, (((1,), (1,)), ((), ())),
                         preferred_element_type=jnp.float32) + bvc_ref[...]
    vtb = vt.astype(jnp.bfloat16)
    pad = jnp.where(
        lax.broadcasted_iota(jnp.int32, (8, _N), 0) == 0,
        1.0, 0.0).astype(jnp.bfloat16)
    for h in range(_H):
        vt_ref[0, pl.ds(h * 72, _HD), :] = vtb[h * _HD:(h + 1) * _HD, :]
        vt_ref[0, pl.ds(h * 72 + _HD, 8), :] = pad


def _proj(x, wqt, bq, wk, bkc, wv, bvc):
    return pl.pallas_call(
        _proj_body,
        grid=(_B,),
        in_specs=[
            pl.BlockSpec((1, _N, _D), lambda b: (b, 0, 0)),
            pl.BlockSpec((_D, _D), lambda b: (0, 0)),
            pl.BlockSpec((1, _D), lambda b: (0, 0)),
            pl.BlockSpec((_D, _D), lambda b: (0, 0)),
            pl.BlockSpec((_D, 1), lambda b: (0, 0)),
            pl.BlockSpec((_D, _D), lambda b: (0, 0)),
            pl.BlockSpec((_D, 1), lambda b: (0, 0)),
        ],
        out_specs=[
            pl.BlockSpec((1, _N, _D), lambda b: (b, 0, 0)),
            pl.BlockSpec((1, _D, _N), lambda b: (b, 0, 0)),
            pl.BlockSpec((1, _H * 72, _N), lambda b: (b, 0, 0)),
        ],
        out_shape=[
            jax.ShapeDtypeStruct((_B, _N, _D), jnp.bfloat16),
            jax.ShapeDtypeStruct((_B, _D, _N), jnp.bfloat16),
            jax.ShapeDtypeStruct((_B, _H * 72, _N), jnp.bfloat16),
        ],
    )(x, wqt, bq, wk, bkc, wv, bvc)


def _topo_body(a_ref, x_ref, w1t_ref, b1_ref, w2_ref, s_ref):
    xb = x_ref[0]
    ab = a_ref[...].astype(jnp.bfloat16)
    x1 = xb.astype(jnp.bfloat16)
    r1 = xb - x1.astype(jnp.float32)
    x2 = r1.astype(jnp.bfloat16)
    x3 = (r1 - x2.astype(jnp.float32)).astype(jnp.bfloat16)
    aggr = (
        jnp.dot(ab, x1, preferred_element_type=jnp.float32)
        + jnp.dot(ab, x2, preferred_element_type=jnp.float32)
        + jnp.dot(ab, x3, preferred_element_type=jnp.float32)
    )
    h = jnp.maximum(
        jnp.dot(aggr, w1t_ref[...], preferred_element_type=jnp.float32)
        + b1_ref[...],
        0.0,
    )
    s_ref[0] = lax.dot_general(
        w2_ref[...], h, (((1,), (1,)), ((), ())),
        preferred_element_type=jnp.float32,
    )


def _topo(a, x, w1t, b1r, w2):
    return pl.pallas_call(
        _topo_body,
        grid=(_B,),
        in_specs=[
            pl.BlockSpec((_N, _N), lambda b: (0, 0)),
            pl.BlockSpec((1, _N, _D), lambda b: (b, 0, 0)),
            pl.BlockSpec((_D, _HID), lambda b: (0, 0)),
            pl.BlockSpec((1, _HID), lambda b: (0, 0)),
            pl.BlockSpec((1, _HID), lambda b: (0, 0)),
        ],
        out_specs=pl.BlockSpec((1, 1, _N), lambda b: (b, 0, 0)),
        out_shape=jax.ShapeDtypeStruct((_B, 1, _N), jnp.float32),
    )(a, x, w1t, b1r, w2)


def _colmask_body(sr_ref, sc_ref, m_ref):
    sr = sr_ref[0]
    sc = sc_ref[0]
    ii = lax.broadcasted_iota(jnp.int32, (_N, _N), 0)
    jj = lax.broadcasted_iota(jnp.int32, (_N, _N), 1)
    gt = (sc > sr).astype(jnp.float32)
    eqb = ((sc == sr) & (ii < jj)).astype(jnp.float32)
    rank = jnp.sum(gt + eqb, axis=0, keepdims=True)
    m_ref[0] = (rank < float(_KTOP)).astype(jnp.float32)


def _colmask(scores_row, scores_col):
    return pl.pallas_call(
        _colmask_body,
        grid=(_B,),
        in_specs=[
            pl.BlockSpec((1, 1, _N), lambda b: (b, 0, 0)),
            pl.BlockSpec((1, _N, 1), lambda b: (b, 0, 0)),
        ],
        out_specs=pl.BlockSpec((1, 1, _N), lambda b: (b, 0, 0)),
        out_shape=jax.ShapeDtypeStruct((_B, 1, _N), jnp.float32),
    )(scores_row, scores_col)


def _attn_body(q_ref, kt_ref, vt_ref, a_ref, cm_ref, wot_ref, bo_ref,
               out_ref, mask_ref):
    edge = (a_ref[...] > 0.0).astype(jnp.float32)
    mask = jnp.maximum(edge, cm_ref[0])
    mask_bf = mask.astype(jnp.bfloat16)

    outs = []
    for h in range(_H):
        qh = q_ref[0][:, h * _HD:(h + 1) * _HD]
        kth = kt_ref[0, pl.ds(h * _HD, _HD), :]
        vth = vt_ref[0, pl.ds(h * 72, 72), :]
        s = jnp.dot(qh, kth, preferred_element_type=jnp.float32)
        em = jnp.exp(s).astype(jnp.bfloat16) * mask_bf
        ohz = lax.dot_general(em, vth, (((1,), (1,)), ((), ())),
                              preferred_element_type=jnp.float32)
        outs.append(ohz[:, :_HD] * (1.0 / ohz[:, _HD:_HD + 1]))
    o = jnp.concatenate(outs, axis=1)
    out_ref[0] = (
        jnp.dot(o, wot_ref[...], preferred_element_type=jnp.float32)
        + bo_ref[...]
    )
    mask_ref[0] = jnp.broadcast_to(mask[None, :, :], (_H, _NQ, _N))


def _attention(q, kt, vt, a, cm, wot, bo):
    return pl.pallas_call(
        _attn_body,
        grid=(_B, _N // _NQ),
        in_specs=[
            pl.BlockSpec((1, _NQ, _D), lambda b, i: (b, i, 0)),
            pl.BlockSpec((1, _D, _N), lambda b, i: (b, 0, 0)),
            pl.BlockSpec((1, _H * 72, _N), lambda b, i: (b, 0, 0)),
            pl.BlockSpec((_NQ, _N), lambda b, i: (i, 0)),
            pl.BlockSpec((1, 1, _N), lambda b, i: (b, 0, 0)),
            pl.BlockSpec((_D, _D), lambda b, i: (0, 0)),
            pl.BlockSpec((1, _D), lambda b, i: (0, 0)),
        ],
        out_specs=[
            pl.BlockSpec((1, _NQ, _D), lambda b, i: (b, i, 0)),
            pl.BlockSpec((1, _H, _NQ, _N), lambda b, i: (b, 0, i, 0)),
        ],
        out_shape=[
            jax.ShapeDtypeStruct((_B, _N, _D), jnp.float32),
            jax.ShapeDtypeStruct((_B, _H, _N, _N), jnp.float32),
        ],
    )(q, kt, vt, a, cm, wot, bo)


def kernel(x, edge_index, Wq, bq, Wk, bk, Wv, bv, Wo, bo, W1, b1, W2, b2):
    a = _build_counts(edge_index[0], edge_index[1])
    q, kt, vt = _proj(
        x,
        Wq.T, bq.reshape(1, _D),
        Wk, bk.reshape(_D, 1),
        Wv, bv.reshape(_D, 1),
    )
    scores = _topo(a, x, W1.T, b1.reshape(1, _HID), W2)
    cm = _colmask(scores, scores.reshape(_B, _N, 1))
    out, mask_h = _attention(q, kt, vt, a, cm, Wo.T, bo.reshape(1, _D))
    return out, mask_h
